# Initial kernel scaffold; baseline (speedup 1.0000x reference)
import jax
import jax.numpy as jnp
from jax.experimental import pallas as pl


def kernel(x, edge_index, w_gate, w_noise, W1, b1, W2, b2):
    raise NotImplementedError("write your pallas kernel here")


# speedup vs baseline: 3.9603x; 3.9603x over previous
import functools

import jax
import jax.numpy as jnp
from jax import lax
from jax.experimental import pallas as pl
from jax.experimental.pallas import tpu as pltpu
from jax.experimental.pallas import tpu_sc as plsc

N_NODES = 10000
N_PAD = 10240
ROWS_PER_SUB = 640
E_EDGES = 160000
E_PAD = 161792
CHUNK = 128
CHUNKS_PER_SUB = 79
D_IN = 256
D_HID = 512
N_EXP = 8
TOPK = 4
COEF = 0.01
NOISE_EPS = 0.01
BN = 400


def _make_prop(P):
    mesh = plsc.VectorSubcoreMesh(core_axis_name="c", subcore_axis_name="s")

    @functools.partial(
        pl.kernel,
        mesh=mesh,
        out_type=jax.ShapeDtypeStruct((P, N_PAD, 128), jnp.float32),
        scratch_types=[
            pltpu.VMEM((CHUNK,), jnp.int32),
            pltpu.VMEM((CHUNK,), jnp.int32),
            pltpu.VMEM((CHUNK, 128), jnp.float32),
            pltpu.VMEM_SHARED((N_PAD, 128), jnp.float32),
            pltpu.SemaphoreType.DMA,
        ],
    )
    def prop(intab, srcp, dstp, zrows, out, srcv, dstv, rows, aggsh, sem):
        c = lax.axis_index("c")
        s = lax.axis_index("s")
        epw = CHUNKS_PER_SUB * CHUNK

        def plane_body(pp, carry):
            p = pp * 2 + c

            @pl.when(p < P)
            def _():
                pltpu.sync_copy(zrows, aggsh.at[pl.ds(s * ROWS_PER_SUB, ROWS_PER_SUB)])
                plsc.subcore_barrier()

                def chunk_body(i, carry2):
                    base = s * epw + i * CHUNK
                    pltpu.sync_copy(srcp.at[pl.ds(base, CHUNK)], srcv)
                    pltpu.sync_copy(dstp.at[pl.ds(base, CHUNK)], dstv)
                    off = p * N_NODES
                    for j in range(CHUNK // 16):
                        srcv[pl.ds(j * 16, 16)] = srcv[pl.ds(j * 16, 16)] + off
                    pltpu.async_copy(intab.at[srcv], rows, sem).wait()
                    pltpu.sync_copy(rows, aggsh.at[dstv], add=True)
                    return carry2

                lax.fori_loop(0, CHUNKS_PER_SUB, chunk_body, 0)
                plsc.subcore_barrier()
                for r in range(ROWS_PER_SUB // CHUNK):
                    rbase = s * ROWS_PER_SUB + r * CHUNK
                    pltpu.sync_copy(aggsh.at[pl.ds(rbase, CHUNK)],
                                    out.at[p, pl.ds(rbase, CHUNK)])
                plsc.subcore_barrier()

            return carry

        lax.fori_loop(0, (P + 1) // 2, plane_body, 0)

    return prop


def _prop(intab, srcp, dstp, P):
    zrows = jnp.zeros((ROWS_PER_SUB, 128), jnp.float32)
    return _make_prop(P)(intab, srcp, dstp, zrows)


def _softplus(v):
    return jnp.maximum(v, 0.0) + jnp.log(1.0 + jnp.exp(-jnp.abs(v)))


def _ncdf(v):
    return 0.5 * (1.0 + lax.erf(v / jnp.sqrt(2.0).astype(v.dtype)))


def _gate_body(x_ref, wg_ref, wn_ref, noise_ref, deg_ref,
               gates_ref, m1_ref, dis_ref, dis128_ref, acc_ref, loss_ref):
    pid = pl.program_id(0)
    x = x_ref[...]
    deg = deg_ref[...] + 1.0
    d_is = lax.rsqrt(deg)

    clean = jnp.dot(x, wg_ref[...], preferred_element_type=jnp.float32)
    raw = jnp.dot(x, wn_ref[...], preferred_element_type=jnp.float32)
    std = _softplus(raw) + NOISE_EPS
    nl = clean + noise_ref[...] * std

    neg = jnp.float32(-1e30)
    v = nl
    for _ in range(TOPK - 1):
        m = jnp.max(v, axis=1, keepdims=True)
        v = jnp.where(v >= m, neg, v)
    thr_out = jnp.max(v, axis=1, keepdims=True)
    v = jnp.where(v >= thr_out, neg, v)
    thr_in = jnp.max(v, axis=1, keepdims=True)

    sel = nl >= thr_out
    rowmax = jnp.max(nl, axis=1, keepdims=True)
    ex = jnp.where(sel, jnp.exp(nl - rowmax), 0.0)
    gates = ex / jnp.sum(ex, axis=1, keepdims=True)
    gates_ref[...] = gates

    m1_ref[...] = x * d_is
    dis_ref[...] = d_is
    dis128_ref[...] = jnp.broadcast_to(d_is, dis128_ref.shape)

    is_in = nl > thr_in
    p_in = _ncdf((clean - thr_in) / std)
    p_out = _ncdf((clean - thr_out) / std)
    load_part = jnp.sum(jnp.where(is_in, p_in, p_out), axis=0, keepdims=True)
    imp_part = jnp.sum(gates, axis=0, keepdims=True)
    part = jnp.concatenate([imp_part, load_part], axis=0)

    @pl.when(pid == 0)
    def _():
        acc_ref[...] = jnp.zeros_like(acc_ref)
        loss_ref[...] = jnp.zeros_like(loss_ref)

    acc_ref[...] += part

    @pl.when(pid == pl.num_programs(0) - 1)
    def _():
        acc = acc_ref[...]

        def cv2(row):
            mu = jnp.mean(row)
            var = jnp.sum((row - mu) ** 2) / (N_EXP - 1)
            return var / (mu * mu + 1e-10)

        loss_ref[...] = jnp.reshape((cv2(acc[0]) + cv2(acc[1])) * COEF, (1, 1))


def _gating(x, w_gate, w_noise, noise, deg):
    nb = N_NODES // BN
    return pl.pallas_call(
        _gate_body,
        grid=(nb,),
        in_specs=[
            pl.BlockSpec((BN, D_IN), lambda i: (i, 0)),
            pl.BlockSpec((D_IN, N_EXP), lambda i: (0, 0)),
            pl.BlockSpec((D_IN, N_EXP), lambda i: (0, 0)),
            pl.BlockSpec((BN, N_EXP), lambda i: (i, 0)),
            pl.BlockSpec((BN, 1), lambda i: (i, 0)),
        ],
        out_specs=[
            pl.BlockSpec((BN, N_EXP), lambda i: (i, 0)),
            pl.BlockSpec((BN, D_IN), lambda i: (i, 0)),
            pl.BlockSpec((BN, 1), lambda i: (i, 0)),
            pl.BlockSpec((BN, 128), lambda i: (i, 0)),
            pl.BlockSpec((2, N_EXP), lambda i: (0, 0)),
            pl.BlockSpec((1, 1), lambda i: (0, 0)),
        ],
        out_shape=[
            jax.ShapeDtypeStruct((N_NODES, N_EXP), jnp.float32),
            jax.ShapeDtypeStruct((N_NODES, D_IN), jnp.float32),
            jax.ShapeDtypeStruct((N_NODES, 1), jnp.float32),
            jax.ShapeDtypeStruct((N_NODES, 128), jnp.float32),
            jax.ShapeDtypeStruct((2, N_EXP), jnp.float32),
            jax.ShapeDtypeStruct((1, 1), jnp.float32),
        ],
    )(x, w_gate, w_noise, noise, deg)


def _expert_body(aggx_ref, m1_ref, t_ref, dis_ref, w1_ref, b1_ref,
                 w2_ref, b2_ref, z_ref):
    d_is = dis_ref[...]
    xp = (aggx_ref[...] + m1_ref[...]) * d_is
    cvec = (t_ref[...] + d_is) * d_is
    h = jnp.dot(xp, w1_ref[0], preferred_element_type=jnp.float32)
    h = jnp.maximum(h + cvec * b1_ref[0], 0.0)
    z = jnp.dot(h, w2_ref[0], preferred_element_type=jnp.float32)
    z = (z + b2_ref[0]) * d_is
    for q in range(4):
        z_ref[0, q] = z[:, q * 128:(q + 1) * 128]


def _experts(aggx, m1, t, dis, W1, b1, W2, b2):
    nb = N_NODES // BN
    return pl.pallas_call(
        _expert_body,
        grid=(N_EXP, nb),
        in_specs=[
            pl.BlockSpec((BN, D_IN), lambda e, i: (i, 0)),
            pl.BlockSpec((BN, D_IN), lambda e, i: (i, 0)),
            pl.BlockSpec((BN, 1), lambda e, i: (i, 0)),
            pl.BlockSpec((BN, 1), lambda e, i: (i, 0)),
            pl.BlockSpec((1, D_IN, D_HID), lambda e, i: (e, 0, 0)),
            pl.BlockSpec((1, 1, D_HID), lambda e, i: (e, 0, 0)),
            pl.BlockSpec((1, D_HID, D_HID), lambda e, i: (e, 0, 0)),
            pl.BlockSpec((1, 1, D_HID), lambda e, i: (e, 0, 0)),
        ],
        out_specs=pl.BlockSpec((1, 4, BN, 128), lambda e, i: (e, 0, i, 0)),
        out_shape=jax.ShapeDtypeStruct((N_EXP, 4, N_NODES, 128), jnp.float32),
    )(aggx, m1, t, dis, W1, b1.reshape(N_EXP, 1, D_HID),
      W2, b2.reshape(N_EXP, 1, D_HID))


def _combine_body(aggz_ref, z_ref, gates_ref, dis_ref, y_ref):
    g = gates_ref[...]
    d_is = dis_ref[...]
    for q in range(4):
        acc = jnp.zeros((BN, 128), jnp.float32)
        for e in range(N_EXP):
            acc += g[:, e:e + 1] * (aggz_ref[e, q] + z_ref[e, q])
        y_ref[:, q * 128:(q + 1) * 128] = acc * d_is


def _combine(aggz, z, gates, dis):
    nb = N_NODES // BN
    return pl.pallas_call(
        _combine_body,
        grid=(nb,),
        in_specs=[
            pl.BlockSpec((N_EXP, 4, BN, 128), lambda i: (0, 0, i, 0)),
            pl.BlockSpec((N_EXP, 4, BN, 128), lambda i: (0, 0, i, 0)),
            pl.BlockSpec((BN, N_EXP), lambda i: (i, 0)),
            pl.BlockSpec((BN, 1), lambda i: (i, 0)),
        ],
        out_specs=pl.BlockSpec((BN, D_HID), lambda i: (i, 0)),
        out_shape=jax.ShapeDtypeStruct((N_NODES, D_HID), jnp.float32),
    )(aggz, z, gates, dis)


def kernel(x, edge_index, w_gate, w_noise, W1, b1, W2, b2):
    src = edge_index[0]
    dst = edge_index[1]
    pad = E_PAD - E_EDGES
    srcp = jnp.concatenate([src, jnp.zeros((pad,), jnp.int32)])
    dstp = jnp.concatenate([dst, jnp.full((pad,), N_NODES, jnp.int32)])

    noise = jax.random.normal(jax.random.key(1), (N_NODES, N_EXP), jnp.float32)

    ones_tab = jnp.ones((N_NODES, 128), jnp.float32)
    deg = _prop(ones_tab, srcp, dstp, 1)[0, :N_NODES, :1]

    gates, m1, dis, dis128, _, loss = _gating(x, w_gate, w_noise, noise, deg)

    m1_planes = m1.reshape(N_NODES, 2, 128).transpose(1, 0, 2).reshape(-1, 128)
    tab1 = jnp.concatenate([m1_planes, dis128], axis=0)
    agg1 = _prop(tab1, srcp, dstp, 3)
    aggx = agg1[:2, :N_NODES].transpose(1, 0, 2).reshape(N_NODES, D_IN)
    t = agg1[2, :N_NODES, :1]

    z = _experts(aggx, m1, t, dis, W1, b1, W2, b2)

    agg2 = _prop(z.reshape(-1, 128), srcp, dstp, 32)
    aggz = agg2.reshape(N_EXP, 4, N_PAD, 128)[:, :, :N_NODES]

    y = _combine(aggz, z, gates, dis)
    return y, loss[0, 0]


# speedup vs baseline: 4.0481x; 1.0222x over previous
import functools

import jax
import jax.numpy as jnp
from jax import lax
from jax.experimental import pallas as pl
from jax.experimental.pallas import tpu as pltpu
from jax.experimental.pallas import tpu_sc as plsc

N_NODES = 10000
N_PAD = 10240
ROWS_PER_SUB = 640
E_EDGES = 160000
E_PAD = 163840
CHUNK = 128
CHUNKS_PER_SUB = 80
D_IN = 256
D_HID = 512
N_EXP = 8
TOPK = 4
COEF = 0.01
NOISE_EPS = 0.01
BN = 400


def _make_prop(P):
    mesh = plsc.VectorSubcoreMesh(core_axis_name="c", subcore_axis_name="s")

    @functools.partial(
        pl.kernel,
        mesh=mesh,
        out_type=jax.ShapeDtypeStruct((P, N_PAD, 128), jnp.float32),
        scratch_types=[
            pltpu.VMEM((CHUNK,), jnp.int32),
            pltpu.VMEM((CHUNK,), jnp.int32),
            pltpu.VMEM((CHUNK, 128), jnp.float32),
            pltpu.VMEM((CHUNK,), jnp.int32),
            pltpu.VMEM((CHUNK,), jnp.int32),
            pltpu.VMEM((CHUNK, 128), jnp.float32),
            pltpu.VMEM_SHARED((N_PAD, 128), jnp.float32),
            pltpu.SemaphoreType.DMA,
            pltpu.SemaphoreType.DMA,
        ],
    )
    def prop(intab, srcp, dstp, zrows, out,
             srcva, dstva, rowsa, srcvb, dstvb, rowsb, aggsh, sema, semb):
        c = lax.axis_index("c")
        s = lax.axis_index("s")
        epw = CHUNKS_PER_SUB * CHUNK

        def plane_body(pp, carry):
            p = pp * 2 + c

            @pl.when(p < P)
            def _():
                pltpu.sync_copy(zrows, aggsh.at[pl.ds(s * ROWS_PER_SUB, ROWS_PER_SUB)])
                plsc.subcore_barrier()
                off = p * N_NODES

                def fire(i, srcv, dstv, rows, sem):
                    base = s * epw + i * CHUNK
                    pltpu.sync_copy(srcp.at[pl.ds(base, CHUNK)], srcv)
                    pltpu.sync_copy(dstp.at[pl.ds(base, CHUNK)], dstv)
                    for j in range(CHUNK // 16):
                        srcv[pl.ds(j * 16, 16)] = srcv[pl.ds(j * 16, 16)] + off
                    return pltpu.async_copy(intab.at[srcv], rows, sem)

                def drain(srcv, rows, dstv, sem):
                    pltpu.make_async_copy(intab.at[srcv], rows, sem).wait()
                    pltpu.sync_copy(rows, aggsh.at[dstv], add=True)

                fire(0, srcva, dstva, rowsa, sema)

                def pair_body(ii, carry2):
                    fire(2 * ii + 1, srcvb, dstvb, rowsb, semb)
                    drain(srcva, rowsa, dstva, sema)

                    @pl.when(2 * ii + 2 < CHUNKS_PER_SUB)
                    def _():
                        fire(2 * ii + 2, srcva, dstva, rowsa, sema)

                    drain(srcvb, rowsb, dstvb, semb)
                    return carry2

                lax.fori_loop(0, CHUNKS_PER_SUB // 2, pair_body, 0)
                plsc.subcore_barrier()
                for r in range(ROWS_PER_SUB // CHUNK):
                    rbase = s * ROWS_PER_SUB + r * CHUNK
                    pltpu.sync_copy(aggsh.at[pl.ds(rbase, CHUNK)],
                                    out.at[p, pl.ds(rbase, CHUNK)])
                plsc.subcore_barrier()

            return carry

        lax.fori_loop(0, (P + 1) // 2, plane_body, 0)

    return prop


def _prop(intab, srcp, dstp, P):
    zrows = jnp.zeros((ROWS_PER_SUB, 128), jnp.float32)
    return _make_prop(P)(intab, srcp, dstp, zrows)


def _softplus(v):
    return jnp.maximum(v, 0.0) + jnp.log(1.0 + jnp.exp(-jnp.abs(v)))


def _ncdf(v):
    return 0.5 * (1.0 + lax.erf(v / jnp.sqrt(2.0).astype(v.dtype)))


def _gate_body(x_ref, wg_ref, wn_ref, noise_ref, deg_ref,
               gates_ref, m1_ref, dis_ref, dis128_ref, acc_ref, loss_ref):
    pid = pl.program_id(0)
    x = x_ref[...]
    deg = deg_ref[...] + 1.0
    d_is = lax.rsqrt(deg)

    clean = jnp.dot(x, wg_ref[...], preferred_element_type=jnp.float32)
    raw = jnp.dot(x, wn_ref[...], preferred_element_type=jnp.float32)
    std = _softplus(raw) + NOISE_EPS
    nl = clean + noise_ref[...] * std

    neg = jnp.float32(-1e30)
    v = nl
    for _ in range(TOPK - 1):
        m = jnp.max(v, axis=1, keepdims=True)
        v = jnp.where(v >= m, neg, v)
    thr_out = jnp.max(v, axis=1, keepdims=True)
    v = jnp.where(v >= thr_out, neg, v)
    thr_in = jnp.max(v, axis=1, keepdims=True)

    sel = nl >= thr_out
    rowmax = jnp.max(nl, axis=1, keepdims=True)
    ex = jnp.where(sel, jnp.exp(nl - rowmax), 0.0)
    gates = ex / jnp.sum(ex, axis=1, keepdims=True)
    gates_ref[...] = gates

    m1_ref[...] = x * d_is
    dis_ref[...] = d_is
    dis128_ref[...] = jnp.broadcast_to(d_is, dis128_ref.shape)

    is_in = nl > thr_in
    p_in = _ncdf((clean - thr_in) / std)
    p_out = _ncdf((clean - thr_out) / std)
    load_part = jnp.sum(jnp.where(is_in, p_in, p_out), axis=0, keepdims=True)
    imp_part = jnp.sum(gates, axis=0, keepdims=True)
    part = jnp.concatenate([imp_part, load_part], axis=0)

    @pl.when(pid == 0)
    def _():
        acc_ref[...] = jnp.zeros_like(acc_ref)
        loss_ref[...] = jnp.zeros_like(loss_ref)

    acc_ref[...] += part

    @pl.when(pid == pl.num_programs(0) - 1)
    def _():
        acc = acc_ref[...]

        def cv2(row):
            mu = jnp.mean(row)
            var = jnp.sum((row - mu) ** 2) / (N_EXP - 1)
            return var / (mu * mu + 1e-10)

        loss_ref[...] = jnp.reshape((cv2(acc[0]) + cv2(acc[1])) * COEF, (1, 1))


def _gating(x, w_gate, w_noise, noise, deg):
    nb = N_NODES // BN
    return pl.pallas_call(
        _gate_body,
        grid=(nb,),
        in_specs=[
            pl.BlockSpec((BN, D_IN), lambda i: (i, 0)),
            pl.BlockSpec((D_IN, N_EXP), lambda i: (0, 0)),
            pl.BlockSpec((D_IN, N_EXP), lambda i: (0, 0)),
            pl.BlockSpec((BN, N_EXP), lambda i: (i, 0)),
            pl.BlockSpec((BN, 1), lambda i: (i, 0)),
        ],
        out_specs=[
            pl.BlockSpec((BN, N_EXP), lambda i: (i, 0)),
            pl.BlockSpec((BN, D_IN), lambda i: (i, 0)),
            pl.BlockSpec((BN, 1), lambda i: (i, 0)),
            pl.BlockSpec((BN, 128), lambda i: (i, 0)),
            pl.BlockSpec((2, N_EXP), lambda i: (0, 0)),
            pl.BlockSpec((1, 1), lambda i: (0, 0)),
        ],
        out_shape=[
            jax.ShapeDtypeStruct((N_NODES, N_EXP), jnp.float32),
            jax.ShapeDtypeStruct((N_NODES, D_IN), jnp.float32),
            jax.ShapeDtypeStruct((N_NODES, 1), jnp.float32),
            jax.ShapeDtypeStruct((N_NODES, 128), jnp.float32),
            jax.ShapeDtypeStruct((2, N_EXP), jnp.float32),
            jax.ShapeDtypeStruct((1, 1), jnp.float32),
        ],
    )(x, w_gate, w_noise, noise, deg)


def _expert_body(aggx_ref, m1_ref, t_ref, dis_ref, w1_ref, b1_ref,
                 w2_ref, b2_ref, z_ref):
    d_is = dis_ref[...]
    xp = (aggx_ref[...] + m1_ref[...]) * d_is
    cvec = (t_ref[...] + d_is) * d_is
    h = jnp.dot(xp, w1_ref[0], preferred_element_type=jnp.float32)
    h = jnp.maximum(h + cvec * b1_ref[0], 0.0)
    z = jnp.dot(h, w2_ref[0], preferred_element_type=jnp.float32)
    z = (z + b2_ref[0]) * d_is
    for q in range(4):
        z_ref[0, q] = z[:, q * 128:(q + 1) * 128]


def _experts(aggx, m1, t, dis, W1, b1, W2, b2):
    nb = N_NODES // BN
    return pl.pallas_call(
        _expert_body,
        grid=(N_EXP, nb),
        in_specs=[
            pl.BlockSpec((BN, D_IN), lambda e, i: (i, 0)),
            pl.BlockSpec((BN, D_IN), lambda e, i: (i, 0)),
            pl.BlockSpec((BN, 1), lambda e, i: (i, 0)),
            pl.BlockSpec((BN, 1), lambda e, i: (i, 0)),
            pl.BlockSpec((1, D_IN, D_HID), lambda e, i: (e, 0, 0)),
            pl.BlockSpec((1, 1, D_HID), lambda e, i: (e, 0, 0)),
            pl.BlockSpec((1, D_HID, D_HID), lambda e, i: (e, 0, 0)),
            pl.BlockSpec((1, 1, D_HID), lambda e, i: (e, 0, 0)),
        ],
        out_specs=pl.BlockSpec((1, 4, BN, 128), lambda e, i: (e, 0, i, 0)),
        out_shape=jax.ShapeDtypeStruct((N_EXP, 4, N_NODES, 128), jnp.float32),
    )(aggx, m1, t, dis, W1, b1.reshape(N_EXP, 1, D_HID),
      W2, b2.reshape(N_EXP, 1, D_HID))


def _combine_body(aggz_ref, z_ref, gates_ref, dis_ref, y_ref):
    g = gates_ref[...]
    d_is = dis_ref[...]
    for q in range(4):
        acc = jnp.zeros((BN, 128), jnp.float32)
        for e in range(N_EXP):
            acc += g[:, e:e + 1] * (aggz_ref[e, q] + z_ref[e, q])
        y_ref[:, q * 128:(q + 1) * 128] = acc * d_is


def _combine(aggz, z, gates, dis):
    nb = N_NODES // BN
    return pl.pallas_call(
        _combine_body,
        grid=(nb,),
        in_specs=[
            pl.BlockSpec((N_EXP, 4, BN, 128), lambda i: (0, 0, i, 0)),
            pl.BlockSpec((N_EXP, 4, BN, 128), lambda i: (0, 0, i, 0)),
            pl.BlockSpec((BN, N_EXP), lambda i: (i, 0)),
            pl.BlockSpec((BN, 1), lambda i: (i, 0)),
        ],
        out_specs=pl.BlockSpec((BN, D_HID), lambda i: (i, 0)),
        out_shape=jax.ShapeDtypeStruct((N_NODES, D_HID), jnp.float32),
    )(aggz, z, gates, dis)


def kernel(x, edge_index, w_gate, w_noise, W1, b1, W2, b2):
    src = edge_index[0]
    dst = edge_index[1]
    pad = E_PAD - E_EDGES
    srcp = jnp.concatenate([src, jnp.zeros((pad,), jnp.int32)])
    dstp = jnp.concatenate([dst, jnp.full((pad,), N_NODES, jnp.int32)])

    noise = jax.random.normal(jax.random.key(1), (N_NODES, N_EXP), jnp.float32)

    ones_tab = jnp.ones((N_NODES, 128), jnp.float32)
    deg = _prop(ones_tab, srcp, dstp, 1)[0, :N_NODES, :1]

    gates, m1, dis, dis128, _, loss = _gating(x, w_gate, w_noise, noise, deg)

    m1_planes = m1.reshape(N_NODES, 2, 128).transpose(1, 0, 2).reshape(-1, 128)
    tab1 = jnp.concatenate([m1_planes, dis128], axis=0)
    agg1 = _prop(tab1, srcp, dstp, 3)
    aggx = agg1[:2, :N_NODES].transpose(1, 0, 2).reshape(N_NODES, D_IN)
    t = agg1[2, :N_NODES, :1]

    z = _experts(aggx, m1, t, dis, W1, b1, W2, b2)

    agg2 = _prop(z.reshape(-1, 128), srcp, dstp, 32)
    aggz = agg2.reshape(N_EXP, 4, N_PAD, 128)[:, :, :N_NODES]

    y = _combine(aggz, z, gates, dis)
    return y, loss[0, 0]
